# Initial kernel scaffold; baseline (speedup 1.0000x reference)
import jax
import jax.numpy as jnp
from jax.experimental import pallas as pl


def kernel(x, edge_index, W_in, b_in, W1, b1, W2, b2, W_out, b_out):
    raise NotImplementedError("write your pallas kernel here")


# speedup vs baseline: 9.1873x; 9.1873x over previous
import functools

import jax
import jax.numpy as jnp
from jax import lax
from jax.experimental import pallas as pl
from jax.experimental.pallas import tpu as pltpu
from jax.experimental.pallas import tpu_sc as plsc

N = 10000
E = 160000
D = 256
HALF = 128
NS = 16
EPT = E // NS
B = 128
NB = (EPT + B - 1) // B
EPAD = NB * B
NP = 10112
RPT = NP // NS

_f32 = jnp.float32


def _mesh():
    return plsc.VectorSubcoreMesh(core_axis_name="c", subcore_axis_name="s")


def _sc_degree(dstp, zeros128, ones128):
    HB = NB // 2

    @functools.partial(
        pl.kernel,
        out_type=jax.ShapeDtypeStruct((2, NP, HALF), _f32),
        mesh=_mesh(),
        scratch_types=[
            pltpu.VMEM((NB, 1, B), jnp.int32),
            pltpu.VMEM((B, HALF), _f32),
            pltpu.VMEM_SHARED((NP, HALF), _f32),
        ],
    )
    def k(dp, z, o, out, dstv, onesv, deg):
        c = lax.axis_index("c")
        s = lax.axis_index("s")
        pltpu.sync_copy(dp.at[s], dstv)
        pltpu.sync_copy(o, onesv)
        pltpu.sync_copy(z.at[pl.ds(s * RPT, RPT)],
                        deg.at[pl.ds(s * RPT, RPT)])
        plsc.subcore_barrier()

        def blk(j, carry):
            pltpu.sync_copy(onesv, deg.at[dstv.at[j, 0]], add=True)
            return carry

        lax.fori_loop(0, NB, blk, 0)
        plsc.subcore_barrier()
        pltpu.sync_copy(deg.at[pl.ds(s * RPT, RPT)],
                        out.at[c].at[pl.ds(s * RPT, RPT)])

    return k(dstp, zeros128, ones128)


def _sc_scatter(g_lo, g_hi, srcp, dstp):
    @functools.partial(
        pl.kernel,
        out_type=(jax.ShapeDtypeStruct((NP, HALF), _f32),
                  jax.ShapeDtypeStruct((NP, HALF), _f32)),
        mesh=_mesh(),
        scratch_types=[
            pltpu.VMEM((NB, 1, B), jnp.int32),
            pltpu.VMEM((NB, 1, B), jnp.int32),
            pltpu.VMEM((B, HALF), _f32),
            pltpu.VMEM_SHARED((NP, HALF), _f32),
            pltpu.SemaphoreType.DMA,
        ],
    )
    def k(glo, ghi, sp, dp, olo, ohi, srcv, dstv, rows, acc, sem):
        c = lax.axis_index("c")
        s = lax.axis_index("s")
        pltpu.sync_copy(sp.at[s], srcv)
        pltpu.sync_copy(dp.at[s], dstv)

        def body(gref, oref):
            pltpu.sync_copy(gref.at[pl.ds(s * RPT, RPT)],
                            acc.at[pl.ds(s * RPT, RPT)])
            plsc.subcore_barrier()

            def blk(j, carry):
                pltpu.async_copy(gref.at[srcv.at[j, 0]], rows, sem).wait()
                pltpu.sync_copy(rows, acc.at[dstv.at[j, 0]], add=True)
                return carry

            lax.fori_loop(0, NB, blk, 0)
            plsc.subcore_barrier()
            pltpu.sync_copy(acc.at[pl.ds(s * RPT, RPT)],
                            oref.at[pl.ds(s * RPT, RPT)])

        @pl.when(c == 0)
        def _():
            body(glo, olo)

        @pl.when(c == 1)
        def _():
            body(ghi, ohi)

    return k(g_lo, g_hi, srcp, dstp)


_BM = 1264


def _leaky(v):
    return jnp.where(v > 0, v, 0.01 * v)


def _dinv(deg_ref):
    return lax.rsqrt(0.5 * (deg_ref[0, :, 0:1] + deg_ref[1, :, 0:1]) + 1.0)


def _deg_spec():
    return pl.BlockSpec((2, _BM, HALF), lambda i: (0, i, 0))


def _tc1_body(x_ref, winT_ref, bin_ref, w1T_ref, deg_ref, glo_ref, ghi_ref):
    h0 = jnp.dot(x_ref[...], winT_ref[...], preferred_element_type=_f32,
                 precision=lax.Precision.HIGHEST)
    h0 = _leaky(h0 + bin_ref[...])
    dinv = _dinv(deg_ref)
    g = dinv * jnp.dot(h0, w1T_ref[...], preferred_element_type=_f32,
                          precision=lax.Precision.HIGHEST)
    glo_ref[...] = g[:, :HALF]
    ghi_ref[...] = g[:, HALF:]


def _tc_mid_body(lo_ref, hi_ref, deg_ref, b_ref, wT_ref, glo_ref, ghi_ref):
    acc = jnp.concatenate([lo_ref[...], hi_ref[...]], axis=1)
    dinv = _dinv(deg_ref)
    h = _leaky(dinv * acc + b_ref[...])
    g = dinv * jnp.dot(h, wT_ref[...], preferred_element_type=_f32,
                          precision=lax.Precision.HIGHEST)
    glo_ref[...] = g[:, :HALF]
    ghi_ref[...] = g[:, HALF:]


def _tc3_body(lo_ref, hi_ref, deg_ref, b_ref, wout_ref, bout_ref, out_ref):
    acc = jnp.concatenate([lo_ref[...], hi_ref[...]], axis=1)
    dinv = _dinv(deg_ref)
    h = _leaky(dinv * acc + b_ref[...])
    out_ref[...] = (jnp.sum(h * wout_ref[...], axis=1, keepdims=True)
                    + bout_ref[...])


def _row_spec(w):
    return pl.BlockSpec((_BM, w), lambda i: (i, 0))


def _full_spec(h, w):
    return pl.BlockSpec((h, w), lambda i: (0, 0))


def _tc1(x, winT, b_in, w1T, deg):
    return pl.pallas_call(
        _tc1_body,
        grid=(NP // _BM,),
        in_specs=[_row_spec(D), _full_spec(D, D), _full_spec(1, D),
                  _full_spec(D, D), _deg_spec()],
        out_specs=[_row_spec(HALF), _row_spec(HALF)],
        out_shape=[jax.ShapeDtypeStruct((NP, HALF), _f32)] * 2,
    )(x, winT, b_in, w1T, deg)


def _tc_mid(lo, hi, deg, b, wT):
    return pl.pallas_call(
        _tc_mid_body,
        grid=(NP // _BM,),
        in_specs=[_row_spec(HALF), _row_spec(HALF), _deg_spec(),
                  _full_spec(1, D), _full_spec(D, D)],
        out_specs=[_row_spec(HALF), _row_spec(HALF)],
        out_shape=[jax.ShapeDtypeStruct((NP, HALF), _f32)] * 2,
    )(lo, hi, deg, b, wT)


def _tc3(lo, hi, deg, b, wout, bout):
    return pl.pallas_call(
        _tc3_body,
        grid=(NP // _BM,),
        in_specs=[_row_spec(HALF), _row_spec(HALF), _deg_spec(),
                  _full_spec(1, D), _full_spec(1, D), _full_spec(1, 1)],
        out_specs=pl.BlockSpec((_BM, 1), lambda i: (i, 0)),
        out_shape=jax.ShapeDtypeStruct((NP, 1), _f32),
    )(lo, hi, deg, b, wout, bout)


def kernel(x, edge_index, W_in, b_in, W1, b1, W2, b2, W_out, b_out):
    src = edge_index[0].astype(jnp.int32)
    dst = edge_index[1].astype(jnp.int32)
    srcp = jnp.pad(src.reshape(NS, EPT), ((0, 0), (0, EPAD - EPT)),
                   constant_values=0).reshape(NS, NB, 1, B)
    dstp = jnp.pad(dst.reshape(NS, EPT), ((0, 0), (0, EPAD - EPT)),
                   constant_values=N).reshape(NS, NB, 1, B)

    deg = _sc_degree(dstp, jnp.zeros((NP, HALF), _f32),
                     jnp.ones((B, HALF), _f32))

    winT = W_in.T
    w1T = W1.T
    w2T = W2.T
    b_in2 = b_in.reshape(1, D)
    b12 = b1.reshape(1, D)
    b22 = b2.reshape(1, D)
    wout2 = W_out.reshape(1, D)
    bout2 = b_out.reshape(1, 1)

    xp = jnp.pad(x, ((0, NP - N), (0, 0)))
    g1lo, g1hi = _tc1(xp, winT, b_in2, w1T, deg)
    a1lo, a1hi = _sc_scatter(g1lo, g1hi, srcp, dstp)
    g2lo, g2hi = _tc_mid(a1lo, a1hi, deg, b12, w2T)
    a2lo, a2hi = _sc_scatter(g2lo, g2hi, srcp, dstp)
    return _tc3(a2lo, a2hi, deg, b22, wout2, bout2)[:N]
